# Initial kernel scaffold; baseline (speedup 1.0000x reference)
import jax
import jax.numpy as jnp
from jax.experimental import pallas as pl


def kernel(x, e_, b_):
    raise NotImplementedError("write your pallas kernel here")


# speedup vs baseline: 23.8212x; 23.8212x over previous
import functools

import jax
import jax.numpy as jnp
from jax import lax
from jax.experimental import pallas as pl
from jax.experimental.pallas import tpu as pltpu
from jax.experimental.pallas import tpu_sc as plsc

_M = 16
_B = 8
_N = 1250
_BN = _B * _N
_D = 3
_E = 160000
_KM_ITERS = 20
_TOPK = 4

_NW = 32
_CHUNK = 128
_NCHUNK = 40
_EPW = _CHUNK * _NCHUNK
_EPAD = _NW * _EPW
_DUMMY_ROW = _BN
_ACC_ROWS = _BN + _M
_STRIPE = 624
_ZTAIL = _ACC_ROWS - 15 * _STRIPE
_CTAIL = _BN - 15 * _STRIPE


def _pool_body(x3t_ref, f_ref, cinit_ref, cent_ref, xp_ref, srow_ref, scol_ref):
    x3 = x3t_ref[0]
    cent0 = cinit_ref[0]

    def dist(cent):
        d = None
        for c in range(_D):
            diff = x3[c:c + 1, :] - cent[:, c:c + 1]
            sq = diff * diff
            d = sq if d is None else d + sq
        return d

    miota = lax.broadcasted_iota(jnp.int32, (_M, _N), 0)

    def step(_, cent):
        d = dist(cent)
        dmin = jnp.min(d, axis=0, keepdims=True)
        first = jnp.min(jnp.where(d == dmin, miota, _M), axis=0, keepdims=True)
        p = (miota == first).astype(jnp.float32)
        sums = lax.dot_general(p, x3, (((1,), (1,)), ((), ())),
                               precision=lax.Precision.HIGHEST)
        cnt = jnp.sum(p, axis=1, keepdims=True)
        mean = sums / jnp.maximum(cnt, 1.0)
        return jnp.where(cnt > 0, mean, cent)

    cent = lax.fori_loop(0, _KM_ITERS, step, cent0)

    s = -dist(cent)
    smax = jnp.max(s, axis=0, keepdims=True)
    e = jnp.exp(s - smax)
    S = e / jnp.sum(e, axis=0, keepdims=True)

    cent_ref[0] = cent
    xp_ref[0] = lax.dot_general(S, f_ref[0], (((1,), (0,)), ((), ())),
                                precision=lax.Precision.HIGHEST)
    scol_ref[0] = S
    srow_ref[0] = S.T


def _pool_call(x3t, f, cinit):
    nf = f.shape[2]
    return pl.pallas_call(
        _pool_body,
        grid=(_B,),
        in_specs=[
            pl.BlockSpec((1, _D, _N), lambda b: (b, 0, 0)),
            pl.BlockSpec((1, _N, nf), lambda b: (b, 0, 0)),
            pl.BlockSpec((1, _M, _D), lambda b: (b, 0, 0)),
        ],
        out_specs=[
            pl.BlockSpec((1, _M, _D), lambda b: (b, 0, 0)),
            pl.BlockSpec((1, _M, nf), lambda b: (b, 0, 0)),
            pl.BlockSpec((1, _N, _M), lambda b: (b, 0, 0)),
            pl.BlockSpec((1, _M, _N), lambda b: (b, 0, 0)),
        ],
        out_shape=[
            jax.ShapeDtypeStruct((_B, _M, _D), jnp.float32),
            jax.ShapeDtypeStruct((_B, _M, nf), jnp.float32),
            jax.ShapeDtypeStruct((_B, _N, _M), jnp.float32),
            jax.ShapeDtypeStruct((_B, _M, _N), jnp.float32),
        ],
    )(x3t, f, cinit)


def _scatter_call(s_rows, e0p, e1p, zrows):
    mesh = plsc.VectorSubcoreMesh(core_axis_name="c", subcore_axis_name="s")

    @functools.partial(
        pl.kernel,
        out_type=jax.ShapeDtypeStruct((2 * _BN, _M), jnp.float32),
        mesh=mesh,
        scratch_types=[
            pltpu.VMEM((_CHUNK,), jnp.int32),
            pltpu.VMEM((_CHUNK,), jnp.int32),
            pltpu.VMEM((_CHUNK, _M), jnp.float32),
            pltpu.VMEM_SHARED((_ACC_ROWS, _M), jnp.float32),
            pltpu.SemaphoreType.DMA,
        ],
        compiler_params=pltpu.CompilerParams(use_tc_tiling_on_sc=False),
    )
    def scatter_kernel(s_hbm, e0_hbm, e1_hbm, z_hbm, out_hbm,
                       idx0_v, idx1_v, rows_v, acc_sh, sem):
        cid = lax.axis_index("c")
        sid = lax.axis_index("s")
        wid = sid * 2 + cid

        @pl.when(sid < 15)
        def _():
            pltpu.sync_copy(z_hbm.at[pl.ds(0, _STRIPE)],
                            acc_sh.at[pl.ds(sid * _STRIPE, _STRIPE)])

        @pl.when(sid == 15)
        def _():
            pltpu.sync_copy(z_hbm, acc_sh.at[pl.ds(15 * _STRIPE, _ZTAIL)])

        plsc.subcore_barrier()

        ebase = wid * _EPW

        def chunk(j, carry):
            b = ebase + j * _CHUNK
            pltpu.sync_copy(e1_hbm.at[pl.ds(b, _CHUNK)], idx1_v)
            pltpu.async_copy(s_hbm.at[idx1_v], rows_v, sem).wait()
            pltpu.sync_copy(e0_hbm.at[pl.ds(b, _CHUNK)], idx0_v)
            pltpu.sync_copy(rows_v, acc_sh.at[idx0_v], add=True)
            return carry

        lax.fori_loop(0, _NCHUNK, chunk, 0)
        plsc.subcore_barrier()

        @pl.when(sid < 15)
        def _():
            pltpu.sync_copy(acc_sh.at[pl.ds(sid * _STRIPE, _STRIPE)],
                            out_hbm.at[pl.ds(cid * _BN + sid * _STRIPE, _STRIPE)])

        @pl.when(sid == 15)
        def _():
            pltpu.sync_copy(acc_sh.at[pl.ds(15 * _STRIPE, _CTAIL)],
                            out_hbm.at[pl.ds(cid * _BN + 15 * _STRIPE, _CTAIL)])

    return scatter_kernel(s_rows, e0p, e1p, zrows)


def _amm_body(scol_ref, as0_ref, as1_ref, dst_ref):
    S = scol_ref[0]
    asb = as0_ref[0] + as1_ref[0]
    a = lax.dot_general(S, asb, (((1,), (0,)), ((), ())),
                        precision=lax.Precision.HIGHEST)
    liota = lax.broadcasted_iota(jnp.int32, (_M, _M), 1)
    cols = []
    for _ in range(_TOPK):
        vmax = jnp.max(a, axis=1, keepdims=True)
        first = jnp.min(jnp.where(a == vmax, liota, _M), axis=1, keepdims=True)
        cols.append(first)
        a = jnp.where(liota == first, -jnp.inf, a)
    dst_ref[0] = jnp.concatenate(cols, axis=1)


def _amm_call(s_cols, as0, as1):
    return pl.pallas_call(
        _amm_body,
        grid=(_B,),
        in_specs=[
            pl.BlockSpec((1, _M, _N), lambda b: (b, 0, 0)),
            pl.BlockSpec((1, _N, _M), lambda b: (b, 0, 0)),
            pl.BlockSpec((1, _N, _M), lambda b: (b, 0, 0)),
        ],
        out_specs=pl.BlockSpec((1, _M, _TOPK), lambda b: (b, 0, 0)),
        out_shape=jax.ShapeDtypeStruct((_B, _M, _TOPK), jnp.int32),
    )(s_cols, as0, as1)


def kernel(x, e_, b_):
    nc = x.shape[1]
    x3b = x[:, :_D].reshape(_B, _N, _D)
    x3t = x3b.transpose(0, 2, 1)
    cinit = x3b[:, :_M, :]
    f = x[:, _D:].reshape(_B, _N, nc - _D)

    cent, xp, s_rows, s_cols = _pool_call(x3t, f, cinit)

    pad = _EPAD - _E
    e0p = jnp.concatenate([e_[0], jnp.full((pad,), _DUMMY_ROW, jnp.int32)])
    e1p = jnp.concatenate([e_[1], jnp.zeros((pad,), jnp.int32)])
    zrows = jnp.zeros((_ZTAIL, _M), jnp.float32)
    as2 = _scatter_call(s_rows.reshape(_BN, _M), e0p, e1p, zrows)

    dst = _amm_call(s_cols, as2[:_BN].reshape(_B, _N, _M),
                    as2[_BN:].reshape(_B, _N, _M))

    x_out = jnp.concatenate(
        [cent.reshape(_B * _M, _D), xp.reshape(_B * _M, nc - _D)], axis=1)
    off = (jnp.arange(_B, dtype=jnp.int32) * _M).reshape(_B, 1, 1)
    src = jnp.broadcast_to(
        jnp.arange(_M, dtype=jnp.int32)[None, :, None], (_B, _M, _TOPK)) + off
    e_out = jnp.stack([src.reshape(-1), (dst + off).reshape(-1)], axis=0)
    b_out = jnp.repeat(jnp.arange(_B, dtype=jnp.int32), _M)
    return (x_out, e_out, b_out)


# speedup vs baseline: 31.0814x; 1.3048x over previous
import functools

import jax
import jax.numpy as jnp
from jax import lax
from jax.experimental import pallas as pl
from jax.experimental.pallas import tpu as pltpu
from jax.experimental.pallas import tpu_sc as plsc

_M = 16
_B = 8
_N = 1250
_BN = _B * _N
_D = 3
_E = 160000
_KM_ITERS = 20
_TOPK = 4

_NW = 32
_CHUNK = 128
_NCHUNK = 40
_EPW = _CHUNK * _NCHUNK
_EPAD = _NW * _EPW
_NBUF = 4
_DUMMY_ROW = _BN
_ACC_ROWS = _BN + _M
_STRIPE = 624
_ZTAIL = _ACC_ROWS - 15 * _STRIPE
_CTAIL = _BN - 15 * _STRIPE


def _pool_body(x3t_ref, f_ref, cinit_ref, cent_ref, xp_ref, srow_ref, scol_ref):
    x3 = x3t_ref[0]
    cent0 = cinit_ref[0]

    def dist(cent):
        d = None
        for c in range(_D):
            diff = x3[c:c + 1, :] - cent[:, c:c + 1]
            sq = diff * diff
            d = sq if d is None else d + sq
        return d

    miota = lax.broadcasted_iota(jnp.int32, (_M, _N), 0)

    def step(_, cent):
        d = dist(cent)
        dmin = jnp.min(d, axis=0, keepdims=True)
        first = jnp.min(jnp.where(d == dmin, miota, _M), axis=0, keepdims=True)
        p = (miota == first).astype(jnp.float32)
        sums = lax.dot_general(p, x3, (((1,), (1,)), ((), ())),
                               precision=lax.Precision.HIGHEST)
        cnt = jnp.sum(p, axis=1, keepdims=True)
        mean = sums / jnp.maximum(cnt, 1.0)
        return jnp.where(cnt > 0, mean, cent)

    cent = lax.fori_loop(0, _KM_ITERS, step, cent0)

    s = -dist(cent)
    smax = jnp.max(s, axis=0, keepdims=True)
    e = jnp.exp(s - smax)
    S = e / jnp.sum(e, axis=0, keepdims=True)

    cent_ref[0] = cent
    xp_ref[0] = lax.dot_general(S, f_ref[0], (((1,), (0,)), ((), ())),
                                precision=lax.Precision.HIGHEST)
    scol_ref[0] = S
    srow_ref[0] = S.T


def _pool_call(x3t, f, cinit):
    nf = f.shape[2]
    return pl.pallas_call(
        _pool_body,
        grid=(_B,),
        in_specs=[
            pl.BlockSpec((1, _D, _N), lambda b: (b, 0, 0)),
            pl.BlockSpec((1, _N, nf), lambda b: (b, 0, 0)),
            pl.BlockSpec((1, _M, _D), lambda b: (b, 0, 0)),
        ],
        out_specs=[
            pl.BlockSpec((1, _M, _D), lambda b: (b, 0, 0)),
            pl.BlockSpec((1, _M, nf), lambda b: (b, 0, 0)),
            pl.BlockSpec((1, _N, _M), lambda b: (b, 0, 0)),
            pl.BlockSpec((1, _M, _N), lambda b: (b, 0, 0)),
        ],
        out_shape=[
            jax.ShapeDtypeStruct((_B, _M, _D), jnp.float32),
            jax.ShapeDtypeStruct((_B, _M, nf), jnp.float32),
            jax.ShapeDtypeStruct((_B, _N, _M), jnp.float32),
            jax.ShapeDtypeStruct((_B, _M, _N), jnp.float32),
        ],
    )(x3t, f, cinit)


def _scatter_call(s_rows, e0p, e1p, zrows):
    mesh = plsc.VectorSubcoreMesh(core_axis_name="c", subcore_axis_name="s")

    @functools.partial(
        pl.kernel,
        out_type=jax.ShapeDtypeStruct((2 * _BN, _M), jnp.float32),
        mesh=mesh,
        scratch_types=[
            pltpu.VMEM((_NCHUNK, _CHUNK), jnp.int32),
            pltpu.VMEM((_NCHUNK, _CHUNK), jnp.int32),
            pltpu.VMEM((_NBUF, _CHUNK, _M), jnp.float32),
            pltpu.VMEM_SHARED((_ACC_ROWS, _M), jnp.float32),
            pltpu.SemaphoreType.DMA,
        ],
        compiler_params=pltpu.CompilerParams(use_tc_tiling_on_sc=False),
    )
    def scatter_kernel(s_hbm, e0_hbm, e1_hbm, z_hbm, out_hbm,
                       idx0_v, idx1_v, rows_v, acc_sh, sem):
        cid = lax.axis_index("c")
        sid = lax.axis_index("s")
        wid = sid * 2 + cid

        @pl.when(sid < 15)
        def _():
            pltpu.sync_copy(z_hbm.at[pl.ds(0, _STRIPE)],
                            acc_sh.at[pl.ds(sid * _STRIPE, _STRIPE)])

        @pl.when(sid == 15)
        def _():
            pltpu.sync_copy(z_hbm, acc_sh.at[pl.ds(15 * _STRIPE, _ZTAIL)])

        plsc.subcore_barrier()

        pltpu.sync_copy(e0_hbm.at[wid], idx0_v)
        pltpu.sync_copy(e1_hbm.at[wid], idx1_v)

        def gather(j, b):
            return pltpu.make_async_copy(
                s_hbm.at[idx1_v.at[j]], rows_v.at[b], sem)

        for b in range(_NBUF):
            gather(b, b).start()

        def group(g, carry):
            for b in range(_NBUF):
                j = g * _NBUF + b
                gather(j, b).wait()
                pltpu.sync_copy(rows_v.at[b], acc_sh.at[idx0_v.at[j]],
                                add=True)

                @pl.when(j + _NBUF < _NCHUNK)
                def _():
                    gather(j + _NBUF, b).start()
            return carry

        lax.fori_loop(0, _NCHUNK // _NBUF, group, 0)
        plsc.subcore_barrier()

        @pl.when(sid < 15)
        def _():
            pltpu.sync_copy(acc_sh.at[pl.ds(sid * _STRIPE, _STRIPE)],
                            out_hbm.at[pl.ds(cid * _BN + sid * _STRIPE, _STRIPE)])

        @pl.when(sid == 15)
        def _():
            pltpu.sync_copy(acc_sh.at[pl.ds(15 * _STRIPE, _CTAIL)],
                            out_hbm.at[pl.ds(cid * _BN + 15 * _STRIPE, _CTAIL)])

    return scatter_kernel(s_rows, e0p, e1p, zrows)


def _amm_body(scol_ref, as0_ref, as1_ref, dst_ref):
    S = scol_ref[0]
    asb = as0_ref[0] + as1_ref[0]
    a = lax.dot_general(S, asb, (((1,), (0,)), ((), ())),
                        precision=lax.Precision.HIGHEST)
    liota = lax.broadcasted_iota(jnp.int32, (_M, _M), 1)
    cols = []
    for _ in range(_TOPK):
        vmax = jnp.max(a, axis=1, keepdims=True)
        first = jnp.min(jnp.where(a == vmax, liota, _M), axis=1, keepdims=True)
        cols.append(first)
        a = jnp.where(liota == first, -jnp.inf, a)
    dst_ref[0] = jnp.concatenate(cols, axis=1)


def _amm_call(s_cols, as0, as1):
    return pl.pallas_call(
        _amm_body,
        grid=(_B,),
        in_specs=[
            pl.BlockSpec((1, _M, _N), lambda b: (b, 0, 0)),
            pl.BlockSpec((1, _N, _M), lambda b: (b, 0, 0)),
            pl.BlockSpec((1, _N, _M), lambda b: (b, 0, 0)),
        ],
        out_specs=pl.BlockSpec((1, _M, _TOPK), lambda b: (b, 0, 0)),
        out_shape=jax.ShapeDtypeStruct((_B, _M, _TOPK), jnp.int32),
    )(s_cols, as0, as1)


def kernel(x, e_, b_):
    nc = x.shape[1]
    x3b = x[:, :_D].reshape(_B, _N, _D)
    x3t = x3b.transpose(0, 2, 1)
    cinit = x3b[:, :_M, :]
    f = x[:, _D:].reshape(_B, _N, nc - _D)

    cent, xp, s_rows, s_cols = _pool_call(x3t, f, cinit)

    pad = _EPAD - _E
    e0p = jnp.concatenate(
        [e_[0], jnp.full((pad,), _DUMMY_ROW, jnp.int32)]
    ).reshape(_NW, _NCHUNK, _CHUNK)
    e1p = jnp.concatenate(
        [e_[1], jnp.zeros((pad,), jnp.int32)]
    ).reshape(_NW, _NCHUNK, _CHUNK)
    zrows = jnp.zeros((_ZTAIL, _M), jnp.float32)
    as2 = _scatter_call(s_rows.reshape(_BN, _M), e0p, e1p, zrows)

    dst = _amm_call(s_cols, as2[:_BN].reshape(_B, _N, _M),
                    as2[_BN:].reshape(_B, _N, _M))

    x_out = jnp.concatenate(
        [cent.reshape(_B * _M, _D), xp.reshape(_B * _M, nc - _D)], axis=1)
    off = (jnp.arange(_B, dtype=jnp.int32) * _M).reshape(_B, 1, 1)
    src = jnp.broadcast_to(
        jnp.arange(_M, dtype=jnp.int32)[None, :, None], (_B, _M, _TOPK)) + off
    e_out = jnp.stack([src.reshape(-1), (dst + off).reshape(-1)], axis=0)
    b_out = jnp.repeat(jnp.arange(_B, dtype=jnp.int32), _M)
    return (x_out, e_out, b_out)
